# baseline (speedup 1.0000x reference)
import jax
import jax.numpy as jnp
from jax import lax
from jax.experimental import pallas as pl
from jax.experimental.pallas import tpu as pltpu
from jax.experimental.pallas import tpu_sc as plsc

N_CLS = 10000
N_CTX = 16
D = 512
SEQ = 77
SUF = 60
B = 1024

NC = 2
NS = 16
NW = NC * NS
BPW = B // NW
NCH = D // 16
HEAD = 1 + N_CTX
SPLIT = 24


def _sc_body(idx_hbm, ctx_hbm, pre_hbm, suf_hbm, out_hbm,
             idx_v, ctx_v, plane_v, sufv, sufw, comboa, gsem, wsem):
    wid = lax.axis_index("s") * NC + lax.axis_index("c")
    base = wid * BPW
    pltpu.sync_copy(idx_hbm.at[pl.ds(base, BPW)], idx_v)
    pltpu.sync_copy(ctx_hbm, ctx_v)

    def place_ctx(r, carry):
        for c in range(NCH):
            comboa[0, 1 + r, pl.ds(c * 16, 16)] = ctx_v[r, pl.ds(c * 16, 16)]
        return carry

    lax.fori_loop(0, N_CTX, place_ctx, 0)

    vec0 = idx_v[pl.ds(0, 16)]
    vec1 = idx_v[pl.ds(16, 16)]
    lanes = lax.iota(jnp.int32, 16)

    def body(i, carry):
        v0, v1 = carry
        sel = jnp.where(i < 16, v0, v1)
        s = jnp.sum(jnp.where(lanes == (i % 16), sel, 0))
        gp = pltpu.async_copy(pre_hbm.at[pl.ds(s // 8, 1)], plane_v, gsem)
        gs = pltpu.async_copy(suf_hbm.at[pl.ds(s, 1)], sufv, gsem)
        gp.wait()
        gs.wait()

        srow = s % 8
        for c in range(NCH):
            comboa[0, 0, pl.ds(c * 16, 16)] = plane_v[0, srow, pl.ds(c * 16, 16)]

        def place_head(r, c2):
            for c in range(NCH):
                comboa[0, HEAD + r, pl.ds(c * 16, 16)] = sufv[0, r, pl.ds(c * 16, 16)]
            return c2

        lax.fori_loop(0, SPLIT - HEAD, place_head, 0)

        def place_tail(r, c2):
            for c in range(NCH):
                sufw[0, 8 + r, pl.ds(c * 16, 16)] = sufv[0, 7 + r, pl.ds(c * 16, 16)]
            return c2

        lax.fori_loop(0, SUF - (SPLIT - HEAD), place_tail, 0)

        b = base + i
        wa = pltpu.async_copy(comboa, out_hbm.at[pl.ds(b, 1), pl.ds(0, SPLIT)], wsem)
        wb = pltpu.async_copy(
            sufw.at[:, pl.ds(8, SEQ - SPLIT)],
            out_hbm.at[pl.ds(b, 1), pl.ds(SPLIT, SEQ - SPLIT)], wsem)
        wa.wait()
        wb.wait()
        return carry

    lax.fori_loop(0, BPW, body, (vec0, vec1))


@jax.jit
def _launch(idx, ctx, pre8, token_suffix):
    call = pl.kernel(
        _sc_body,
        out_type=jax.ShapeDtypeStruct((B, SEQ, D), jnp.float32),
        mesh=plsc.VectorSubcoreMesh(core_axis_name="c", subcore_axis_name="s"),
        compiler_params=pltpu.CompilerParams(needs_layout_passes=False),
        scratch_types=[
            pltpu.VMEM((BPW,), jnp.int32),
            pltpu.VMEM((N_CTX, D), jnp.float32),
            pltpu.VMEM((1, 8, D), jnp.float32),
            pltpu.VMEM((1, SUF, D), jnp.float32),
            pltpu.VMEM((1, SUF + 1, D), jnp.float32),
            pltpu.VMEM((1, SPLIT, D), jnp.float32),
            pltpu.SemaphoreType.DMA,
            pltpu.SemaphoreType.DMA,
        ],
    )
    return call(idx, ctx, pre8, token_suffix)


def kernel(idx, ctx, token_prefix, token_suffix):
    pre8 = token_prefix.reshape(N_CLS // 8, 8, D)
    return _launch(idx, ctx, pre8, token_suffix)
